# Initial kernel scaffold; baseline (speedup 1.0000x reference)
import jax
import jax.numpy as jnp
from jax.experimental import pallas as pl


def kernel(x, edge_index, W1, b1, W2, b2):
    raise NotImplementedError("write your pallas kernel here")


# speedup vs baseline: 16.9273x; 16.9273x over previous
import functools

import jax
import jax.numpy as jnp
from jax import lax
from jax.experimental import pallas as pl
from jax.experimental.pallas import tpu as pltpu
from jax.experimental.pallas import tpu_sc as plsc

N_NODES = 10000
D = 256
HALF = 128
C = 128
DC = 128
NS = 16
NC = 2
EPT = 10240
NCHUNK = EPT // C
EPW = 5120
DCHUNK = EPW // DC
E_PAD = EPT * NS
N_PAD = 10112
RC = 128
RCHUNKS = N_PAD // RC
N_TRASH = N_PAD - N_NODES


def _scatter_body(g0, g1, idx3, o0, o1, acc, idxa, idxb, bufa, bufb,
                  sema, semb, semia, semib):
    c = lax.axis_index("c")
    s = lax.axis_index("s")

    def run(g_hbm, out_hbm):
        for j in range(5):
            k = s * 5 + j

            @pl.when(k < RCHUNKS)
            def _():
                rows = pl.ds(k * RC, RC)
                pltpu.sync_copy(g_hbm.at[rows], acc.at[rows])

        plsc.subcore_barrier()

        def fetch_idx(k, ibuf, sem):
            pltpu.async_copy(idx3.at[s, k], ibuf, sem)

        def wait_idx(k, ibuf, sem):
            pltpu.make_async_copy(idx3.at[s, k], ibuf, sem).wait()

        def gather(ibuf, buf, sem):
            pltpu.async_copy(g_hbm.at[ibuf.at[0]], buf, sem)

        def wait_gather(ibuf, buf, sem):
            pltpu.make_async_copy(g_hbm.at[ibuf.at[0]], buf, sem).wait()

        def scatter(ibuf, buf):
            pltpu.sync_copy(buf, acc.at[ibuf.at[1]], add=True)

        fetch_idx(0, idxa, semia)
        wait_idx(0, idxa, semia)
        gather(idxa, bufa, sema)
        fetch_idx(1, idxb, semib)

        def body(j, carry):
            k0 = 2 * j
            wait_idx(k0 + 1, idxb, semib)
            gather(idxb, bufb, semb)
            wait_gather(idxa, bufa, sema)
            scatter(idxa, bufa)

            @pl.when(j < NCHUNK // 2 - 1)
            def _():
                fetch_idx(k0 + 2, idxa, semia)
                wait_idx(k0 + 2, idxa, semia)
                gather(idxa, bufa, sema)

            wait_gather(idxb, bufb, semb)
            scatter(idxb, bufb)

            @pl.when(j < NCHUNK // 2 - 1)
            def _():
                fetch_idx(k0 + 3, idxb, semib)

            return carry

        lax.fori_loop(0, NCHUNK // 2, body, 0)
        plsc.subcore_barrier()

        for j in range(5):
            k = s * 5 + j

            @pl.when(k < RCHUNKS)
            def _():
                rows = pl.ds(k * RC, RC)
                pltpu.sync_copy(acc.at[rows], out_hbm.at[rows])

    @pl.when(c == 0)
    def _():
        run(g0, o0)

    @pl.when(c == 1)
    def _():
        run(g1, o1)


def _sc_scatter(g0, g1, idx3):
    mesh = plsc.VectorSubcoreMesh(core_axis_name="c", subcore_axis_name="s")
    f = pl.kernel(
        _scatter_body,
        out_type=(
            jax.ShapeDtypeStruct((N_PAD, HALF), jnp.float32),
            jax.ShapeDtypeStruct((N_PAD, HALF), jnp.float32),
        ),
        mesh=mesh,
        scratch_types=[
            pltpu.VMEM_SHARED((N_PAD, HALF), jnp.float32),
            pltpu.VMEM((2, C), jnp.int32),
            pltpu.VMEM((2, C), jnp.int32),
            pltpu.VMEM((C, HALF), jnp.float32),
            pltpu.VMEM((C, HALF), jnp.float32),
            pltpu.SemaphoreType.DMA,
            pltpu.SemaphoreType.DMA,
            pltpu.SemaphoreType.DMA,
            pltpu.SemaphoreType.DMA,
        ],
    )
    return f(g0, g1, idx3)


def _deg_body(dstd, d0, d1, dacc, dstv, ones, zbuf, sem_unused):
    c = lax.axis_index("c")
    s = lax.axis_index("s")
    w = s * NC + c

    pltpu.sync_copy(dstd.at[w], dstv)

    for i in range(DC // 16):
        ones[pl.ds(i * 16, 16)] = jnp.full((16,), 1.0, jnp.float32)
    for i in range(RC // 16):
        zbuf[pl.ds(i * 16, 16)] = jnp.zeros((16,), jnp.float32)
    for j in range(5):
        k = s * 5 + j

        @pl.when(k < RCHUNKS)
        def _():
            pltpu.sync_copy(zbuf, dacc.at[pl.ds(k * RC, RC)])

    plsc.subcore_barrier()

    def body(k, carry):
        pltpu.sync_copy(ones, dacc.at[dstv.at[k]], add=True)
        return carry

    lax.fori_loop(0, DCHUNK, body, 0)
    plsc.subcore_barrier()

    for j in range(5):
        k = s * 5 + j

        @pl.when(k < RCHUNKS)
        def _():
            rows = pl.ds(k * RC, RC)

            @pl.when(c == 0)
            def _():
                pltpu.sync_copy(dacc.at[rows], d0.at[rows])

            @pl.when(c == 1)
            def _():
                pltpu.sync_copy(dacc.at[rows], d1.at[rows])


def _sc_deg(dstd):
    mesh = plsc.VectorSubcoreMesh(core_axis_name="c", subcore_axis_name="s")
    f = pl.kernel(
        _deg_body,
        out_type=(
            jax.ShapeDtypeStruct((N_PAD,), jnp.float32),
            jax.ShapeDtypeStruct((N_PAD,), jnp.float32),
        ),
        mesh=mesh,
        scratch_types=[
            pltpu.VMEM_SHARED((N_PAD,), jnp.float32),
            pltpu.VMEM((DCHUNK, DC), jnp.int32),
            pltpu.VMEM((DC,), jnp.float32),
            pltpu.VMEM((RC,), jnp.float32),
            pltpu.SemaphoreType.DMA,
        ],
    )
    return f(dstd)


ROWB = 632


def _tc1_body(x_ref, w_ref, d0_ref, d1_ref, g0_ref, g1_ref, dinv_ref):
    deg = 1.0 + d0_ref[...] + d1_ref[...]
    dinv = lax.rsqrt(deg)
    h = jnp.dot(x_ref[...], w_ref[...], preferred_element_type=jnp.float32)
    g = h * dinv
    g0_ref[...] = g[:, :HALF]
    g1_ref[...] = g[:, HALF:]
    dinv_ref[...] = dinv


def _tc1(x_pad, W1, d0, d1):
    grid = (N_PAD // ROWB,)
    return pl.pallas_call(
        _tc1_body,
        grid=grid,
        in_specs=[
            pl.BlockSpec((ROWB, D), lambda i: (i, 0)),
            pl.BlockSpec((D, D), lambda i: (0, 0)),
            pl.BlockSpec((ROWB, 1), lambda i: (i, 0)),
            pl.BlockSpec((ROWB, 1), lambda i: (i, 0)),
        ],
        out_specs=(
            pl.BlockSpec((ROWB, HALF), lambda i: (i, 0)),
            pl.BlockSpec((ROWB, HALF), lambda i: (i, 0)),
            pl.BlockSpec((ROWB, 1), lambda i: (i, 0)),
        ),
        out_shape=(
            jax.ShapeDtypeStruct((N_PAD, HALF), jnp.float32),
            jax.ShapeDtypeStruct((N_PAD, HALF), jnp.float32),
            jax.ShapeDtypeStruct((N_PAD, 1), jnp.float32),
        ),
    )(x_pad, W1, d0, d1)


def _tc2_body(s0_ref, s1_ref, dinv_ref, b1_ref, w_ref, g0_ref, g1_ref):
    dinv = dinv_ref[...]
    sfull = jnp.concatenate([s0_ref[...], s1_ref[...]], axis=1)
    t = jnp.maximum(sfull * dinv + b1_ref[...], 0.0)
    g = jnp.dot(t, w_ref[...], preferred_element_type=jnp.float32) * dinv
    g0_ref[...] = g[:, :HALF]
    g1_ref[...] = g[:, HALF:]


def _tc2(s0, s1, dinv, b1, W2):
    grid = (N_PAD // ROWB,)
    return pl.pallas_call(
        _tc2_body,
        grid=grid,
        in_specs=[
            pl.BlockSpec((ROWB, HALF), lambda i: (i, 0)),
            pl.BlockSpec((ROWB, HALF), lambda i: (i, 0)),
            pl.BlockSpec((ROWB, 1), lambda i: (i, 0)),
            pl.BlockSpec((1, D), lambda i: (0, 0)),
            pl.BlockSpec((D, D), lambda i: (0, 0)),
        ],
        out_specs=(
            pl.BlockSpec((ROWB, HALF), lambda i: (i, 0)),
            pl.BlockSpec((ROWB, HALF), lambda i: (i, 0)),
        ),
        out_shape=(
            jax.ShapeDtypeStruct((N_PAD, HALF), jnp.float32),
            jax.ShapeDtypeStruct((N_PAD, HALF), jnp.float32),
        ),
    )(s0, s1, dinv, b1, W2)


def _tc3_body(s0_ref, s1_ref, dinv_ref, b2_ref, o_ref):
    sfull = jnp.concatenate([s0_ref[...], s1_ref[...]], axis=1)
    o_ref[...] = sfull * dinv_ref[...] + b2_ref[...]


def _tc3(s0, s1, dinv, b2):
    grid = (N_PAD // ROWB,)
    return pl.pallas_call(
        _tc3_body,
        grid=grid,
        in_specs=[
            pl.BlockSpec((ROWB, HALF), lambda i: (i, 0)),
            pl.BlockSpec((ROWB, HALF), lambda i: (i, 0)),
            pl.BlockSpec((ROWB, 1), lambda i: (i, 0)),
            pl.BlockSpec((1, D), lambda i: (0, 0)),
        ],
        out_specs=pl.BlockSpec((ROWB, D), lambda i: (i, 0)),
        out_shape=jax.ShapeDtypeStruct((N_PAD, D), jnp.float32),
    )(s0, s1, dinv, b2)


@jax.jit
def _run(x, edge_index, W1, b1, W2, b2):
    src = edge_index[0].astype(jnp.int32)
    dst = edge_index[1].astype(jnp.int32)
    e = src.shape[0]
    npad = E_PAD - e
    pad_src = jnp.arange(npad, dtype=jnp.int32) % N_NODES
    pad_dst = N_NODES + jnp.arange(npad, dtype=jnp.int32) % N_TRASH
    src_p = jnp.concatenate([src, pad_src])
    dst_p = jnp.concatenate([dst, pad_dst])
    idx3 = jnp.stack(
        [src_p.reshape(NS, NCHUNK, C), dst_p.reshape(NS, NCHUNK, C)], axis=2
    )
    dstd = dst_p.reshape(NS * NC, DCHUNK, DC)

    x_pad = jnp.pad(x, ((0, N_PAD - N_NODES), (0, 0)))
    b1r = b1.reshape(1, D)
    b2r = b2.reshape(1, D)

    d0, d1 = _sc_deg(dstd)
    g0, g1, dinv = _tc1(x_pad, W1, d0.reshape(N_PAD, 1), d1.reshape(N_PAD, 1))
    s0, s1 = _sc_scatter(g0, g1, idx3)
    g20, g21 = _tc2(s0, s1, dinv, b1r, W2)
    t0, t1 = _sc_scatter(g20, g21, idx3)
    out = _tc3(t0, t1, dinv, b2r)
    return out[:N_NODES]


def kernel(x, edge_index, W1, b1, W2, b2):
    return _run(x, edge_index, W1, b1, W2, b2)


# speedup vs baseline: 18.6110x; 1.0995x over previous
import functools

import jax
import jax.numpy as jnp
from jax import lax
from jax.experimental import pallas as pl
from jax.experimental.pallas import tpu as pltpu
from jax.experimental.pallas import tpu_sc as plsc

N_NODES = 10000
D = 256
HALF = 128
C = 128
DC = 128
NS = 16
NC = 2
EPT = 10240
NCHUNK = EPT // C
EPW = 5120
DCHUNK = EPW // DC
E_PAD = EPT * NS
N_PAD = 10112
RC = 128
RCHUNKS = N_PAD // RC
N_TRASH = N_PAD - N_NODES


def _scatter_body(g0, g1, idx3, o0, o1, acc, idxa, idxb, idxc, idxd,
                  bufa, bufb, sema, semb, semia, semib, semic, semid):
    c = lax.axis_index("c")
    s = lax.axis_index("s")

    def run(g_hbm, out_hbm):
        for j in range(5):
            k = s * 5 + j

            @pl.when(k < RCHUNKS)
            def _():
                rows = pl.ds(k * RC, RC)
                pltpu.sync_copy(g_hbm.at[rows], acc.at[rows])

        plsc.subcore_barrier()

        def fetch_idx(k, ibuf, sem):
            pltpu.async_copy(idx3.at[s, k], ibuf, sem)

        def wait_idx(k, ibuf, sem):
            pltpu.make_async_copy(idx3.at[s, k], ibuf, sem).wait()

        def gather(ibuf, buf, sem):
            pltpu.async_copy(g_hbm.at[ibuf.at[0]], buf, sem)

        def wait_gather(ibuf, buf, sem):
            pltpu.make_async_copy(g_hbm.at[ibuf.at[0]], buf, sem).wait()

        def scatter(ibuf, buf):
            pltpu.sync_copy(buf, acc.at[ibuf.at[1]], add=True)

        ibufs = (idxa, idxb, idxc, idxd)
        isems = (semia, semib, semic, semid)
        for k in range(4):
            fetch_idx(k, ibufs[k], isems[k])
        wait_idx(0, idxa, semia)
        gather(idxa, bufa, sema)

        def body2(t, carry):
            k0 = 4 * t

            def half(kk, ia, sa, ib, sb, ja, jb, sja, sjb):
                wait_idx(kk + 1, ib, sb)
                gather(ib, bufb, semb)
                wait_gather(ia, bufa, sema)
                scatter(ia, bufa)

                @pl.when(kk + 2 < NCHUNK)
                def _():
                    wait_idx(kk + 2, ja, sja)
                    gather(ja, bufa, sema)

                @pl.when(kk + 4 < NCHUNK)
                def _():
                    fetch_idx(kk + 4, ia, sa)

                wait_gather(ib, bufb, semb)
                scatter(ib, bufb)

                @pl.when(kk + 5 < NCHUNK)
                def _():
                    fetch_idx(kk + 5, ib, sb)

            half(k0, idxa, semia, idxb, semib, idxc, idxd, semic, semid)
            half(k0 + 2, idxc, semic, idxd, semid, idxa, idxb, semia, semib)
            return carry

        lax.fori_loop(0, NCHUNK // 4, body2, 0)
        plsc.subcore_barrier()

        for j in range(5):
            k = s * 5 + j

            @pl.when(k < RCHUNKS)
            def _():
                rows = pl.ds(k * RC, RC)
                pltpu.sync_copy(acc.at[rows], out_hbm.at[rows])

    @pl.when(c == 0)
    def _():
        run(g0, o0)

    @pl.when(c == 1)
    def _():
        run(g1, o1)


def _sc_scatter(g0, g1, idx3):
    mesh = plsc.VectorSubcoreMesh(core_axis_name="c", subcore_axis_name="s")
    f = pl.kernel(
        _scatter_body,
        out_type=(
            jax.ShapeDtypeStruct((N_PAD, HALF), jnp.float32),
            jax.ShapeDtypeStruct((N_PAD, HALF), jnp.float32),
        ),
        mesh=mesh,
        scratch_types=[
            pltpu.VMEM_SHARED((N_PAD, HALF), jnp.float32),
            pltpu.VMEM((2, C), jnp.int32),
            pltpu.VMEM((2, C), jnp.int32),
            pltpu.VMEM((2, C), jnp.int32),
            pltpu.VMEM((2, C), jnp.int32),
            pltpu.VMEM((C, HALF), jnp.float32),
            pltpu.VMEM((C, HALF), jnp.float32),
            pltpu.SemaphoreType.DMA,
            pltpu.SemaphoreType.DMA,
            pltpu.SemaphoreType.DMA,
            pltpu.SemaphoreType.DMA,
            pltpu.SemaphoreType.DMA,
            pltpu.SemaphoreType.DMA,
        ],
    )
    return f(g0, g1, idx3)


def _deg_body(dstd, d0, d1, dacc, dstv, ones, zbuf, sem_unused):
    c = lax.axis_index("c")
    s = lax.axis_index("s")
    w = s * NC + c

    pltpu.sync_copy(dstd.at[w], dstv)

    for i in range(DC // 16):
        ones[pl.ds(i * 16, 16)] = jnp.full((16,), 1.0, jnp.float32)
    for i in range(RC // 16):
        zbuf[pl.ds(i * 16, 16)] = jnp.zeros((16,), jnp.float32)
    for j in range(5):
        k = s * 5 + j

        @pl.when(k < RCHUNKS)
        def _():
            pltpu.sync_copy(zbuf, dacc.at[pl.ds(k * RC, RC)])

    plsc.subcore_barrier()

    def body(k, carry):
        pltpu.sync_copy(ones, dacc.at[dstv.at[k]], add=True)
        return carry

    lax.fori_loop(0, DCHUNK, body, 0)
    plsc.subcore_barrier()

    for j in range(5):
        k = s * 5 + j

        @pl.when(k < RCHUNKS)
        def _():
            rows = pl.ds(k * RC, RC)

            @pl.when(c == 0)
            def _():
                pltpu.sync_copy(dacc.at[rows], d0.at[rows])

            @pl.when(c == 1)
            def _():
                pltpu.sync_copy(dacc.at[rows], d1.at[rows])


def _sc_deg(dstd):
    mesh = plsc.VectorSubcoreMesh(core_axis_name="c", subcore_axis_name="s")
    f = pl.kernel(
        _deg_body,
        out_type=(
            jax.ShapeDtypeStruct((N_PAD,), jnp.float32),
            jax.ShapeDtypeStruct((N_PAD,), jnp.float32),
        ),
        mesh=mesh,
        scratch_types=[
            pltpu.VMEM_SHARED((N_PAD,), jnp.float32),
            pltpu.VMEM((DCHUNK, DC), jnp.int32),
            pltpu.VMEM((DC,), jnp.float32),
            pltpu.VMEM((RC,), jnp.float32),
            pltpu.SemaphoreType.DMA,
        ],
    )
    return f(dstd)


ROWB = 632


def _tc1_body(x_ref, w_ref, d0_ref, d1_ref, g0_ref, g1_ref, dinv_ref):
    deg = 1.0 + d0_ref[...] + d1_ref[...]
    dinv = lax.rsqrt(deg)
    h = jnp.dot(x_ref[...], w_ref[...], preferred_element_type=jnp.float32)
    g = h * dinv
    g0_ref[...] = g[:, :HALF]
    g1_ref[...] = g[:, HALF:]
    dinv_ref[...] = dinv


def _tc1(x_pad, W1, d0, d1):
    grid = (N_PAD // ROWB,)
    return pl.pallas_call(
        _tc1_body,
        grid=grid,
        in_specs=[
            pl.BlockSpec((ROWB, D), lambda i: (i, 0)),
            pl.BlockSpec((D, D), lambda i: (0, 0)),
            pl.BlockSpec((ROWB, 1), lambda i: (i, 0)),
            pl.BlockSpec((ROWB, 1), lambda i: (i, 0)),
        ],
        out_specs=(
            pl.BlockSpec((ROWB, HALF), lambda i: (i, 0)),
            pl.BlockSpec((ROWB, HALF), lambda i: (i, 0)),
            pl.BlockSpec((ROWB, 1), lambda i: (i, 0)),
        ),
        out_shape=(
            jax.ShapeDtypeStruct((N_PAD, HALF), jnp.float32),
            jax.ShapeDtypeStruct((N_PAD, HALF), jnp.float32),
            jax.ShapeDtypeStruct((N_PAD, 1), jnp.float32),
        ),
    )(x_pad, W1, d0, d1)


def _tc2_body(s0_ref, s1_ref, dinv_ref, b1_ref, w_ref, g0_ref, g1_ref):
    dinv = dinv_ref[...]
    sfull = jnp.concatenate([s0_ref[...], s1_ref[...]], axis=1)
    t = jnp.maximum(sfull * dinv + b1_ref[...], 0.0)
    g = jnp.dot(t, w_ref[...], preferred_element_type=jnp.float32) * dinv
    g0_ref[...] = g[:, :HALF]
    g1_ref[...] = g[:, HALF:]


def _tc2(s0, s1, dinv, b1, W2):
    grid = (N_PAD // ROWB,)
    return pl.pallas_call(
        _tc2_body,
        grid=grid,
        in_specs=[
            pl.BlockSpec((ROWB, HALF), lambda i: (i, 0)),
            pl.BlockSpec((ROWB, HALF), lambda i: (i, 0)),
            pl.BlockSpec((ROWB, 1), lambda i: (i, 0)),
            pl.BlockSpec((1, D), lambda i: (0, 0)),
            pl.BlockSpec((D, D), lambda i: (0, 0)),
        ],
        out_specs=(
            pl.BlockSpec((ROWB, HALF), lambda i: (i, 0)),
            pl.BlockSpec((ROWB, HALF), lambda i: (i, 0)),
        ),
        out_shape=(
            jax.ShapeDtypeStruct((N_PAD, HALF), jnp.float32),
            jax.ShapeDtypeStruct((N_PAD, HALF), jnp.float32),
        ),
    )(s0, s1, dinv, b1, W2)


def _tc3_body(s0_ref, s1_ref, dinv_ref, b2_ref, o_ref):
    sfull = jnp.concatenate([s0_ref[...], s1_ref[...]], axis=1)
    o_ref[...] = sfull * dinv_ref[...] + b2_ref[...]


def _tc3(s0, s1, dinv, b2):
    grid = (N_PAD // ROWB,)
    return pl.pallas_call(
        _tc3_body,
        grid=grid,
        in_specs=[
            pl.BlockSpec((ROWB, HALF), lambda i: (i, 0)),
            pl.BlockSpec((ROWB, HALF), lambda i: (i, 0)),
            pl.BlockSpec((ROWB, 1), lambda i: (i, 0)),
            pl.BlockSpec((1, D), lambda i: (0, 0)),
        ],
        out_specs=pl.BlockSpec((ROWB, D), lambda i: (i, 0)),
        out_shape=jax.ShapeDtypeStruct((N_PAD, D), jnp.float32),
    )(s0, s1, dinv, b2)


@jax.jit
def _run(x, edge_index, W1, b1, W2, b2):
    src = edge_index[0].astype(jnp.int32)
    dst = edge_index[1].astype(jnp.int32)
    e = src.shape[0]
    npad = E_PAD - e
    pad_src = jnp.arange(npad, dtype=jnp.int32) % N_NODES
    pad_dst = N_NODES + jnp.arange(npad, dtype=jnp.int32) % N_TRASH
    src_p = jnp.concatenate([src, pad_src])
    dst_p = jnp.concatenate([dst, pad_dst])
    idx3 = jnp.stack(
        [src_p.reshape(NS, NCHUNK, C), dst_p.reshape(NS, NCHUNK, C)], axis=2
    )
    dstd = dst_p.reshape(NS * NC, DCHUNK, DC)

    x_pad = jnp.pad(x, ((0, N_PAD - N_NODES), (0, 0)))
    b1r = b1.reshape(1, D)
    b2r = b2.reshape(1, D)

    d0, d1 = _sc_deg(dstd)
    g0, g1, dinv = _tc1(x_pad, W1, d0.reshape(N_PAD, 1), d1.reshape(N_PAD, 1))
    s0, s1 = _sc_scatter(g0, g1, idx3)
    g20, g21 = _tc2(s0, s1, dinv, b1r, W2)
    t0, t1 = _sc_scatter(g20, g21, idx3)
    out = _tc3(t0, t1, dinv, b2r)
    return out[:N_NODES]


def kernel(x, edge_index, W1, b1, W2, b2):
    return _run(x, edge_index, W1, b1, W2, b2)
